# Initial kernel scaffold; baseline (speedup 1.0000x reference)
import jax
import jax.numpy as jnp
from jax.experimental import pallas as pl


def kernel(x, emb_table):
    raise NotImplementedError("write your pallas kernel here")


# speedup vs baseline: 1.2771x; 1.2771x over previous
import jax
import jax.numpy as jnp
from jax.experimental import pallas as pl


def _add_kernel(x_ref, emb_ref, o_ref):
    o_ref[...] = x_ref[...] + emb_ref[...]


def kernel(x, emb_table):
    B, S, E = x.shape
    BS = 512
    grid = (B, S // BS)
    return pl.pallas_call(
        _add_kernel,
        grid=grid,
        in_specs=[
            pl.BlockSpec((1, BS, E), lambda b, s: (b, s, 0)),
            pl.BlockSpec((BS, E), lambda b, s: (s, 0)),
        ],
        out_specs=pl.BlockSpec((1, BS, E), lambda b, s: (b, s, 0)),
        out_shape=jax.ShapeDtypeStruct((B, S, E), x.dtype),
    )(x, emb_table[:S])


# speedup vs baseline: 1.4858x; 1.1634x over previous
import jax
import jax.numpy as jnp
from jax.experimental import pallas as pl


def _add_kernel(x_ref, emb_ref, o_ref):
    o_ref[...] = x_ref[...] + emb_ref[...]


def kernel(x, emb_table):
    B, S, E = x.shape
    BS = 512
    grid = (S // BS, B)
    return pl.pallas_call(
        _add_kernel,
        grid=grid,
        in_specs=[
            pl.BlockSpec((1, BS, E), lambda s, b: (b, s, 0)),
            pl.BlockSpec((BS, E), lambda s, b: (s, 0)),
        ],
        out_specs=pl.BlockSpec((1, BS, E), lambda s, b: (b, s, 0)),
        out_shape=jax.ShapeDtypeStruct((B, S, E), x.dtype),
    )(x, emb_table[:S])


# speedup vs baseline: 1.6620x; 1.1186x over previous
import jax
import jax.numpy as jnp
from jax.experimental import pallas as pl


def _add_kernel(x_ref, emb_ref, o_ref):
    o_ref[...] = x_ref[...] + emb_ref[...]


def kernel(x, emb_table):
    B, S, E = x.shape
    BS = 1024
    grid = (S // BS, B)
    return pl.pallas_call(
        _add_kernel,
        grid=grid,
        in_specs=[
            pl.BlockSpec((1, BS, E), lambda s, b: (b, s, 0)),
            pl.BlockSpec((BS, E), lambda s, b: (s, 0)),
        ],
        out_specs=pl.BlockSpec((1, BS, E), lambda s, b: (b, s, 0)),
        out_shape=jax.ShapeDtypeStruct((B, S, E), x.dtype),
    )(x, emb_table[:S])


# speedup vs baseline: 1.7399x; 1.0469x over previous
import jax
import jax.numpy as jnp
from jax.experimental import pallas as pl


def _add_kernel(x_ref, emb_ref, o_ref):
    o_ref[...] = x_ref[...] + emb_ref[...]


def kernel(x, emb_table):
    B, S, E = x.shape
    BS = 2048
    grid = (S // BS, B)
    return pl.pallas_call(
        _add_kernel,
        grid=grid,
        in_specs=[
            pl.BlockSpec((1, BS, E), lambda s, b: (b, s, 0)),
            pl.BlockSpec((BS, E), lambda s, b: (s, 0)),
        ],
        out_specs=pl.BlockSpec((1, BS, E), lambda s, b: (b, s, 0)),
        out_shape=jax.ShapeDtypeStruct((B, S, E), x.dtype),
    )(x, emb_table[:S])
